# Initial kernel scaffold; baseline (speedup 1.0000x reference)
import jax
import jax.numpy as jnp
from jax.experimental import pallas as pl


def kernel(x, edge_index, edge_attr, W1, b1, Wa, ba, Wb, bb):
    raise NotImplementedError("write your pallas kernel here")


# speedup vs baseline: 6.3657x; 6.3657x over previous
import functools

import jax
import jax.numpy as jnp
from jax import lax
from jax.experimental import pallas as pl
from jax.experimental.pallas import tpu as pltpu
from jax.experimental.pallas import tpu_sc as plsc

N_NODES = 10000
N_EDGES = 320000
D = 128
EDGE_DIM = 16

NC = 2
NS = 16
NW = NC * NS
CH = 128
NCHUNK = N_EDGES // CH
ITERS = (NCHUNK + NW - 1) // NW
RSUB = 624
TAIL = N_NODES - NS * RSUB

_mesh = plsc.VectorSubcoreMesh(
    core_axis_name="c", subcore_axis_name="s", num_cores=NC, num_subcores=NS)
_sc_params = pltpu.CompilerParams(needs_layout_passes=False)


def _worker_ids():
  cid = lax.axis_index("c")
  sid = lax.axis_index("s")
  return cid, sid, sid * NC + cid


def _deg_body(dst_hbm, degp_hbm, ones_b, zero_b, dst_v, acc):
  cid, sid, wid = _worker_ids()

  one_row = jnp.where(lax.iota(jnp.int32, 16) == 0, 1.0, 0.0)
  zro_row = jnp.zeros((16,), jnp.float32)

  def fill(i, _):
    ones_b[i] = one_row
    zero_b[i] = zro_row
    return _
  lax.fori_loop(0, CH, fill, None)

  for t in range(4):
    pltpu.sync_copy(zero_b, acc.at[pl.ds(sid * RSUB + t * CH, CH)])
  pltpu.sync_copy(zero_b.at[pl.ds(0, RSUB - 4 * CH)],
                  acc.at[pl.ds(sid * RSUB + 4 * CH, RSUB - 4 * CH)])

  @pl.when(sid == 0)
  def _():
    pltpu.sync_copy(zero_b.at[pl.ds(0, TAIL)], acc.at[pl.ds(NS * RSUB, TAIL)])
  plsc.subcore_barrier()

  def step(i, _):
    chunk = wid + NW * i

    @pl.when(chunk < NCHUNK)
    def _():
      pltpu.sync_copy(dst_hbm.at[pl.ds(chunk * CH, CH)], dst_v)
      pltpu.sync_copy(ones_b, acc.at[dst_v], add=True)
    return _
  lax.fori_loop(0, ITERS, step, None)

  plsc.subcore_barrier()
  pltpu.sync_copy(acc.at[pl.ds(sid * RSUB, RSUB)],
                  degp_hbm.at[cid, pl.ds(sid * RSUB, RSUB)])

  @pl.when(sid == 0)
  def _():
    pltpu.sync_copy(acc.at[pl.ds(NS * RSUB, TAIL)],
                    degp_hbm.at[cid, pl.ds(NS * RSUB, TAIL)])


_deg_kernel = pl.kernel(
    _deg_body,
    out_type=jax.ShapeDtypeStruct((NC, N_NODES, 16), jnp.float32),
    mesh=_mesh,
    compiler_params=_sc_params,
    scratch_types=[
        pltpu.VMEM((CH, 16), jnp.float32),
        pltpu.VMEM((CH, 16), jnp.float32),
        pltpu.VMEM((CH,), jnp.int32),
        pltpu.VMEM_SHARED((N_NODES, 16), jnp.float32),
    ],
)


def _k2_body(x_ref, w1_ref, degp_ref, y_ref):
  deg = 1.0 + jnp.sum(degp_ref[...], axis=(0, 2))
  dinv = lax.rsqrt(deg)
  xw = jnp.dot(x_ref[...], w1_ref[...], preferred_element_type=jnp.float32)
  y_ref[...] = xw * dinv[:, None]


def _scatter_body(y_hbm, src_hbm, dst_hbm, aggp_hbm, rows, src_v,
                  dst_v, acc, sem):
  cid, sid, wid = _worker_ids()

  zro = jnp.zeros((16,), jnp.float32)

  def zfill(i, _):
    for k in range(8):
      rows[i, pl.ds(k * 16, 16)] = zro
    return _
  lax.fori_loop(0, CH, zfill, None)

  for t in range(4):
    pltpu.sync_copy(rows, acc.at[pl.ds(sid * RSUB + t * CH, CH)])
  pltpu.sync_copy(rows.at[pl.ds(0, RSUB - 4 * CH)],
                  acc.at[pl.ds(sid * RSUB + 4 * CH, RSUB - 4 * CH)])

  @pl.when(sid == 0)
  def _():
    pltpu.sync_copy(rows.at[pl.ds(0, TAIL)], acc.at[pl.ds(NS * RSUB, TAIL)])
  plsc.subcore_barrier()

  def step(i, _):
    chunk = wid + NW * i

    @pl.when(chunk < NCHUNK)
    def _():
      base = chunk * CH
      pltpu.sync_copy(src_hbm.at[pl.ds(base, CH)], src_v)
      pltpu.sync_copy(dst_hbm.at[pl.ds(base, CH)], dst_v)
      pltpu.async_copy(y_hbm.at[src_v], rows, sem).wait()
      pltpu.sync_copy(rows, acc.at[dst_v], add=True)
    return _
  lax.fori_loop(0, ITERS, step, None)

  plsc.subcore_barrier()
  pltpu.sync_copy(acc.at[pl.ds(sid * RSUB, RSUB)],
                  aggp_hbm.at[cid, pl.ds(sid * RSUB, RSUB)])

  @pl.when(sid == 0)
  def _():
    pltpu.sync_copy(acc.at[pl.ds(NS * RSUB, TAIL)],
                    aggp_hbm.at[cid, pl.ds(NS * RSUB, TAIL)])


_scatter_kernel = pl.kernel(
    _scatter_body,
    out_type=jax.ShapeDtypeStruct((NC, N_NODES, D), jnp.float32),
    mesh=_mesh,
    compiler_params=_sc_params,
    scratch_types=[
        pltpu.VMEM((CH, D), jnp.float32),
        pltpu.VMEM((CH,), jnp.int32),
        pltpu.VMEM((CH,), jnp.int32),
        pltpu.VMEM_SHARED((N_NODES, D), jnp.float32),
        pltpu.SemaphoreType.DMA,
    ],
)


def _k4a_body(aggp_ref, y_ref, degp_ref, b1_ref, wa_ref, a_ref, b_ref):
  deg = 1.0 + jnp.sum(degp_ref[...], axis=(0, 2))
  dinv = lax.rsqrt(deg)
  pre = dinv[:, None] * (aggp_ref[0] + aggp_ref[1] + y_ref[...]) + b1_ref[...]
  h = jnp.maximum(pre, 0.0)
  a_ref[...] = jnp.dot(h, wa_ref[0:D, :], preferred_element_type=jnp.float32)
  b_ref[...] = jnp.dot(h, wa_ref[D:2 * D, :],
                       preferred_element_type=jnp.float32)


def _k4b_body(ea_ref, wa_ref, ba_ref, c_ref):
  c_ref[...] = jnp.dot(ea_ref[...], wa_ref[2 * D:2 * D + EDGE_DIM, :],
                       preferred_element_type=jnp.float32) + ba_ref[...]


def _edge_body(a_hbm, b_hbm, c_hbm, src_hbm, dst_hbm, wb_hbm, bb_hbm,
               out_hbm, a_buf, b_buf, c_buf, src_v, dst_v, wb_v, bb_v, out_v,
               sem_a, sem_b):
  cid, sid, wid = _worker_ids()

  pltpu.sync_copy(wb_hbm, wb_v)
  pltpu.sync_copy(bb_hbm, bb_v)
  wbk = [wb_v[pl.ds(k * 16, 16)] for k in range(8)]
  bbv = bb_v[...]

  def step(i, _):
    chunk = wid + NW * i

    @pl.when(chunk < NCHUNK)
    def _():
      base = chunk * CH
      pltpu.sync_copy(src_hbm.at[pl.ds(base, CH)], src_v)
      pltpu.sync_copy(dst_hbm.at[pl.ds(base, CH)], dst_v)
      ca = pltpu.async_copy(a_hbm.at[src_v], a_buf, sem_a)
      cb = pltpu.async_copy(b_hbm.at[dst_v], b_buf, sem_b)
      pltpu.sync_copy(c_hbm.at[pl.ds(base, CH)], c_buf)
      ca.wait()
      cb.wait()

      lane = lax.iota(jnp.int32, 16)

      def group(g, _):
        out_vec = jnp.zeros((16,), jnp.float32)
        for l in range(16):
          j = g * 16 + l
          acc = jnp.zeros((16,), jnp.float32)
          for k in range(8):
            sl = pl.ds(k * 16, 16)
            t = jnp.maximum(a_buf[j, sl] + b_buf[j, sl] + c_buf[j, sl], 0.0)
            acc = acc + t * wbk[k]
          out_vec = jnp.where(lane == l, jnp.sum(acc), out_vec)
        v = out_vec + bbv
        out_v[pl.ds(g * 16, 16)] = 1.0 / (1.0 + jnp.exp(-v))
        return _
      lax.fori_loop(0, CH // 16, group, None)

      pltpu.sync_copy(out_v, out_hbm.at[pl.ds(base, CH)])
    return _
  lax.fori_loop(0, ITERS, step, None)


_edge_kernel = pl.kernel(
    _edge_body,
    out_type=jax.ShapeDtypeStruct((N_EDGES,), jnp.float32),
    mesh=_mesh,
    compiler_params=_sc_params,
    scratch_types=[
        pltpu.VMEM((CH, D), jnp.float32),
        pltpu.VMEM((CH, D), jnp.float32),
        pltpu.VMEM((CH, D), jnp.float32),
        pltpu.VMEM((CH,), jnp.int32),
        pltpu.VMEM((CH,), jnp.int32),
        pltpu.VMEM((D,), jnp.float32),
        pltpu.VMEM((16,), jnp.float32),
        pltpu.VMEM((CH,), jnp.float32),
        pltpu.SemaphoreType.DMA,
        pltpu.SemaphoreType.DMA,
    ],
)


def kernel(x, edge_index, edge_attr, W1, b1, Wa, ba, Wb, bb):
  src = edge_index[0].astype(jnp.int32)
  dst = edge_index[1].astype(jnp.int32)

  degp = _deg_kernel(dst)

  y = pl.pallas_call(
      _k2_body,
      out_shape=jax.ShapeDtypeStruct((N_NODES, D), jnp.float32),
  )(x, W1, degp)

  aggp = _scatter_kernel(y, src, dst)

  a_tab, b_tab = pl.pallas_call(
      _k4a_body,
      out_shape=(jax.ShapeDtypeStruct((N_NODES, D), jnp.float32),
                 jax.ShapeDtypeStruct((N_NODES, D), jnp.float32)),
  )(aggp, y, degp, b1.reshape(1, D), Wa)

  blk = 20000
  c_tab = pl.pallas_call(
      _k4b_body,
      grid=(N_EDGES // blk,),
      in_specs=[
          pl.BlockSpec((blk, EDGE_DIM), lambda i: (i, 0)),
          pl.BlockSpec((2 * D + EDGE_DIM, D), lambda i: (0, 0)),
          pl.BlockSpec((1, D), lambda i: (0, 0)),
      ],
      out_specs=pl.BlockSpec((blk, D), lambda i: (i, 0)),
      out_shape=jax.ShapeDtypeStruct((N_EDGES, D), jnp.float32),
  )(edge_attr, Wa, ba.reshape(1, D))

  scores = _edge_kernel(a_tab, b_tab, c_tab, src, dst, Wb[:, 0],
                        jnp.broadcast_to(bb, (16,)))
  return scores


# speedup vs baseline: 12.0774x; 1.8973x over previous
import jax
import jax.numpy as jnp
from jax import lax
from jax.experimental import pallas as pl
from jax.experimental.pallas import tpu as pltpu
from jax.experimental.pallas import tpu_sc as plsc

N_NODES = 10000
N_EDGES = 320000
D = 128
EDGE_DIM = 16

NC = 2
NS = 16
NW = NC * NS
CH = 128
NCHUNK = N_EDGES // CH
BASE_CNT = NCHUNK // NW
EXTRA = NCHUNK - BASE_CNT * NW
MAXC = BASE_CNT + 1
RSUB = 624
TAIL = N_NODES - NS * RSUB

_mesh = plsc.VectorSubcoreMesh(
    core_axis_name="c", subcore_axis_name="s", num_cores=NC, num_subcores=NS)
_sc_params = pltpu.CompilerParams(needs_layout_passes=False)


def _worker_ids():
  cid = lax.axis_index("c")
  sid = lax.axis_index("s")
  wid = sid * NC + cid
  start = BASE_CNT * wid + jnp.minimum(wid, EXTRA)
  count = BASE_CNT + jnp.where(wid < EXTRA, 1, 0)
  return cid, sid, start, count


def _zero_spmem_range(zbuf, acc, sid):
  for t in range(4):
    pltpu.sync_copy(zbuf, acc.at[pl.ds(sid * RSUB + t * CH, CH)])
  pltpu.sync_copy(zbuf.at[pl.ds(0, RSUB - 4 * CH)],
                  acc.at[pl.ds(sid * RSUB + 4 * CH, RSUB - 4 * CH)])

  @pl.when(sid == 0)
  def _():
    pltpu.sync_copy(zbuf.at[pl.ds(0, TAIL)], acc.at[pl.ds(NS * RSUB, TAIL)])


def _export_spmem_range(acc, out_hbm, cid, sid):
  pltpu.sync_copy(acc.at[pl.ds(sid * RSUB, RSUB)],
                  out_hbm.at[cid, pl.ds(sid * RSUB, RSUB)])

  @pl.when(sid == 0)
  def _():
    pltpu.sync_copy(acc.at[pl.ds(NS * RSUB, TAIL)],
                    out_hbm.at[cid, pl.ds(NS * RSUB, TAIL)])


def _deg_body(dst_hbm, degp_hbm, ones_b, zero_b, idx_r, acc, sem_i, sem_s):
  cid, sid, start, count = _worker_ids()

  one_row = jnp.where(lax.iota(jnp.int32, 16) == 0, 1.0, 0.0)
  zro_row = jnp.zeros((16,), jnp.float32)

  def fill(i, _):
    ones_b[i] = one_row
    zero_b[i] = zro_row
    return _
  lax.fori_loop(0, CH, fill, None)

  _zero_spmem_range(zero_b, acc, sid)
  plsc.subcore_barrier()

  def _idx(c, slot):
    return (dst_hbm.at[pl.ds((start + c) * CH, CH)], idx_r.at[slot],
            sem_i.at[slot])

  def _sct(slot):
    return (ones_b, acc.at[idx_r.at[slot]], sem_s.at[slot])

  for c in range(4):
    pltpu.async_copy(*_idx(c, c))

  def step(t, _):
    for u in range(8):
      i = 8 * t + u

      @pl.when(i < count)
      def _():
        pltpu.make_async_copy(*_idx(i, u)).wait()
        s, d, m = _sct(u)
        pltpu.async_copy(s, d, m, add=True)

        @pl.when(i >= 4)
        def _():
          s2, d2, m2 = _sct((u + 4) % 8)
          pltpu.make_async_copy(s2, d2, m2).wait()

        @pl.when(i + 4 < count)
        def _():
          pltpu.async_copy(*_idx(i + 4, (u + 4) % 8))
    return _
  lax.fori_loop(0, (MAXC + 7) // 8, step, None)

  for cnt in (BASE_CNT, MAXC):
    @pl.when(count == cnt)
    def _():
      for k in range(1, 5):
        s, d, m = _sct((cnt - k) % 8)
        pltpu.make_async_copy(s, d, m).wait()

  plsc.subcore_barrier()
  _export_spmem_range(acc, degp_hbm, cid, sid)


_deg_kernel = pl.kernel(
    _deg_body,
    out_type=jax.ShapeDtypeStruct((NC, N_NODES, 16), jnp.float32),
    mesh=_mesh,
    compiler_params=_sc_params,
    scratch_types=[
        pltpu.VMEM((CH, 16), jnp.float32),
        pltpu.VMEM((CH, 16), jnp.float32),
        pltpu.VMEM((8, CH), jnp.int32),
        pltpu.VMEM_SHARED((N_NODES, 16), jnp.float32),
        pltpu.SemaphoreType.DMA((8,)),
        pltpu.SemaphoreType.DMA((8,)),
    ],
)


def _k2_body(x_ref, w1_ref, degp_ref, y_ref):
  deg = 1.0 + jnp.sum(degp_ref[...], axis=(0, 2))
  dinv = lax.rsqrt(deg)
  xw = jnp.dot(x_ref[...], w1_ref[...], preferred_element_type=jnp.float32)
  y_ref[...] = xw * dinv[:, None]


def _scatter_body(y_hbm, src_hbm, dst_hbm, aggp_hbm, rows, idx_r, acc,
                  sem_i, sem_g, sem_s):
  cid, sid, start, count = _worker_ids()

  zro = jnp.zeros((16,), jnp.float32)

  def zfill(i, _):
    for k in range(8):
      rows[0, i, pl.ds(k * 16, 16)] = zro
    return _
  lax.fori_loop(0, CH, zfill, None)

  _zero_spmem_range(rows.at[0], acc, sid)
  plsc.subcore_barrier()

  def _idx(c, slot):
    g = (start + c) * CH
    return ((src_hbm.at[pl.ds(g, CH)], idx_r.at[slot, 0], sem_i.at[slot]),
            (dst_hbm.at[pl.ds(g, CH)], idx_r.at[slot, 1], sem_i.at[slot]))

  def _gat(islot, rslot):
    return (y_hbm.at[idx_r.at[islot, 0]], rows.at[rslot], sem_g.at[rslot])

  def _sct(islot, rslot):
    return (rows.at[rslot], acc.at[idx_r.at[islot, 1]], sem_s.at[rslot])

  def fire_idx(c, slot):
    a, b = _idx(c, slot)
    pltpu.async_copy(*a)
    pltpu.async_copy(*b)

  def wait_idx(c, slot):
    a, b = _idx(c, slot)
    pltpu.make_async_copy(*a).wait()
    pltpu.make_async_copy(*b).wait()

  for c in range(4):
    fire_idx(c, c)
  for c in range(2):
    wait_idx(c, c)
    pltpu.async_copy(*_gat(c, c))

  def step(t, _):
    for u in range(6):
      i = 6 * t + u
      ru = u % 3

      @pl.when(i < count)
      def _():
        g = _gat(u, ru)
        pltpu.make_async_copy(*g).wait()
        s, d, m = _sct(u, ru)
        pltpu.async_copy(s, d, m, add=True)

        @pl.when(i >= 1)
        def _():
          s2, d2, m2 = _sct((u + 5) % 6, (ru + 2) % 3)
          pltpu.make_async_copy(s2, d2, m2).wait()

        @pl.when(i + 2 < count)
        def _():
          wait_idx(i + 2, (u + 2) % 6)
          pltpu.async_copy(*_gat((u + 2) % 6, (ru + 2) % 3))

        @pl.when(i + 4 < count)
        def _():
          fire_idx(i + 4, (u + 4) % 6)
    return _
  lax.fori_loop(0, (MAXC + 5) // 6, step, None)

  for cnt in (BASE_CNT, MAXC):
    @pl.when(count == cnt)
    def _():
      s, d, m = _sct((cnt - 1) % 6, (cnt - 1) % 3)
      pltpu.make_async_copy(s, d, m).wait()

  plsc.subcore_barrier()
  _export_spmem_range(acc, aggp_hbm, cid, sid)


_scatter_kernel = pl.kernel(
    _scatter_body,
    out_type=jax.ShapeDtypeStruct((NC, N_NODES, D), jnp.float32),
    mesh=_mesh,
    compiler_params=_sc_params,
    scratch_types=[
        pltpu.VMEM((3, CH, D), jnp.float32),
        pltpu.VMEM((6, 2, CH), jnp.int32),
        pltpu.VMEM_SHARED((N_NODES, D), jnp.float32),
        pltpu.SemaphoreType.DMA((6,)),
        pltpu.SemaphoreType.DMA((3,)),
        pltpu.SemaphoreType.DMA((3,)),
    ],
)


def _k4a_body(aggp_ref, y_ref, degp_ref, b1_ref, wa_ref, a_ref, b_ref):
  deg = 1.0 + jnp.sum(degp_ref[...], axis=(0, 2))
  dinv = lax.rsqrt(deg)
  pre = dinv[:, None] * (aggp_ref[0] + aggp_ref[1] + y_ref[...]) + b1_ref[...]
  h = jnp.maximum(pre, 0.0)
  a_ref[...] = jnp.dot(h, wa_ref[0:D, :], preferred_element_type=jnp.float32)
  b_ref[...] = jnp.dot(h, wa_ref[D:2 * D, :],
                       preferred_element_type=jnp.float32)


def _k4b_body(ea_ref, wa_ref, ba_ref, c_ref):
  c_ref[...] = jnp.dot(ea_ref[...], wa_ref[2 * D:2 * D + EDGE_DIM, :],
                       preferred_element_type=jnp.float32) + ba_ref[...]


def _edge_body(a_hbm, b_hbm, c_hbm, src_hbm, dst_hbm, wb_hbm, bb_hbm,
               out_hbm, a2, b2, c2, idx_r, wb_v, bb_v, out_b,
               sem_a, sem_b, sem_c, sem_i):
  cid, sid, start, count = _worker_ids()

  pltpu.sync_copy(wb_hbm, wb_v)
  pltpu.sync_copy(bb_hbm, bb_v)
  wbk = [wb_v[pl.ds(k * 16, 16)] for k in range(8)]
  bbv = bb_v[...]
  lane = lax.iota(jnp.int32, 16)

  def _idx(c, slot):
    g = (start + c) * CH
    return ((src_hbm.at[pl.ds(g, CH)], idx_r.at[slot, 0], sem_i.at[slot]),
            (dst_hbm.at[pl.ds(g, CH)], idx_r.at[slot, 1], sem_i.at[slot]))

  def _abc(c, islot, p):
    return ((a_hbm.at[idx_r.at[islot, 0]], a2.at[p], sem_a.at[p]),
            (b_hbm.at[idx_r.at[islot, 1]], b2.at[p], sem_b.at[p]),
            (c_hbm.at[pl.ds((start + c) * CH, CH)], c2.at[p], sem_c.at[p]))

  def fire_idx(c, slot):
    a, b = _idx(c, slot)
    pltpu.async_copy(*a)
    pltpu.async_copy(*b)

  def wait_idx(c, slot):
    a, b = _idx(c, slot)
    pltpu.make_async_copy(*a).wait()
    pltpu.make_async_copy(*b).wait()

  def fire_abc(c, islot, p):
    for triple in _abc(c, islot, p):
      pltpu.async_copy(*triple)

  def wait_abc(c, islot, p):
    for triple in _abc(c, islot, p):
      pltpu.make_async_copy(*triple).wait()

  fire_idx(0, 0)
  fire_idx(1, 1)
  wait_idx(0, 0)
  fire_abc(0, 0, 0)
  wait_idx(1, 1)
  fire_abc(1, 1, 1)
  fire_idx(2, 2)
  fire_idx(3, 3)

  def step(t, _):
    for u in range(4):
      i = 4 * t + u
      p = u % 2

      @pl.when(i < count)
      def _():
        wait_abc(i, u, p)

        def group(g, _):
          out_vec = jnp.zeros((16,), jnp.float32)
          for l in range(16):
            j = g * 16 + l
            acc = jnp.zeros((16,), jnp.float32)
            for k in range(8):
              sl = pl.ds(k * 16, 16)
              v = a2[p, j, sl] + b2[p, j, sl] + c2[p, j, sl]
              acc = acc + jnp.maximum(v, 0.0) * wbk[k]
            out_vec = jnp.where(lane == l, jnp.sum(acc), out_vec)
          sv = out_vec + bbv
          out_b[pl.ds(i * CH + g * 16, 16)] = 1.0 / (1.0 + jnp.exp(-sv))
          return _
        lax.fori_loop(0, CH // 16, group, None)

        @pl.when(i + 2 < count)
        def _():
          wait_idx(i + 2, (u + 2) % 4)
          fire_abc(i + 2, (u + 2) % 4, p)

        @pl.when(i + 4 < count)
        def _():
          fire_idx(i + 4, u)
    return _
  lax.fori_loop(0, (MAXC + 3) // 4, step, None)

  pltpu.sync_copy(out_b.at[pl.ds(0, BASE_CNT * CH)],
                  out_hbm.at[pl.ds(start * CH, BASE_CNT * CH)])

  @pl.when(count == MAXC)
  def _():
    pltpu.sync_copy(out_b.at[pl.ds(BASE_CNT * CH, CH)],
                    out_hbm.at[pl.ds((start + BASE_CNT) * CH, CH)])


_edge_kernel = pl.kernel(
    _edge_body,
    out_type=jax.ShapeDtypeStruct((N_EDGES,), jnp.float32),
    mesh=_mesh,
    compiler_params=_sc_params,
    scratch_types=[
        pltpu.VMEM((2, CH, D), jnp.float32),
        pltpu.VMEM((2, CH, D), jnp.float32),
        pltpu.VMEM((2, CH, D), jnp.float32),
        pltpu.VMEM((4, 2, CH), jnp.int32),
        pltpu.VMEM((D,), jnp.float32),
        pltpu.VMEM((16,), jnp.float32),
        pltpu.VMEM((MAXC * CH,), jnp.float32),
        pltpu.SemaphoreType.DMA((2,)),
        pltpu.SemaphoreType.DMA((2,)),
        pltpu.SemaphoreType.DMA((2,)),
        pltpu.SemaphoreType.DMA((4,)),
    ],
)


def kernel(x, edge_index, edge_attr, W1, b1, Wa, ba, Wb, bb):
  src = edge_index[0].astype(jnp.int32)
  dst = edge_index[1].astype(jnp.int32)

  degp = _deg_kernel(dst)

  y = pl.pallas_call(
      _k2_body,
      out_shape=jax.ShapeDtypeStruct((N_NODES, D), jnp.float32),
  )(x, W1, degp)

  aggp = _scatter_kernel(y, src, dst)

  a_tab, b_tab = pl.pallas_call(
      _k4a_body,
      out_shape=(jax.ShapeDtypeStruct((N_NODES, D), jnp.float32),
                 jax.ShapeDtypeStruct((N_NODES, D), jnp.float32)),
  )(aggp, y, degp, b1.reshape(1, D), Wa)

  blk = 20000
  c_tab = pl.pallas_call(
      _k4b_body,
      grid=(N_EDGES // blk,),
      in_specs=[
          pl.BlockSpec((blk, EDGE_DIM), lambda i: (i, 0)),
          pl.BlockSpec((2 * D + EDGE_DIM, D), lambda i: (0, 0)),
          pl.BlockSpec((1, D), lambda i: (0, 0)),
      ],
      out_specs=pl.BlockSpec((blk, D), lambda i: (i, 0)),
      out_shape=jax.ShapeDtypeStruct((N_EDGES, D), jnp.float32),
  )(edge_attr, Wa, ba.reshape(1, D))

  scores = _edge_kernel(a_tab, b_tab, c_tab, src, dst, Wb[:, 0],
                        jnp.broadcast_to(bb, (16,)))
  return scores


# speedup vs baseline: 16.5897x; 1.3736x over previous
import jax
import jax.numpy as jnp
import numpy as np
from jax import lax
from jax.experimental import pallas as pl
from jax.experimental.pallas import tpu as pltpu
from jax.experimental.pallas import tpu_sc as plsc

N_NODES = 10000
N_EDGES = 320000
D = 128
EDGE_DIM = 16

NC = 2
NS = 16
NW = NC * NS
CH = 128
NCHUNK = N_EDGES // CH
BASE_CNT = NCHUNK // NW
EXTRA = NCHUNK - BASE_CNT * NW
MAXC = BASE_CNT + 1
RSUB = 624
TAIL = N_NODES - NS * RSUB

_mesh = plsc.VectorSubcoreMesh(
    core_axis_name="c", subcore_axis_name="s", num_cores=NC, num_subcores=NS)
_sc_params = pltpu.CompilerParams(needs_layout_passes=False)


def _worker_ids():
  cid = lax.axis_index("c")
  sid = lax.axis_index("s")
  wid = sid * NC + cid
  start = BASE_CNT * wid + jnp.minimum(wid, EXTRA)
  count = BASE_CNT + jnp.where(wid < EXTRA, 1, 0)
  return cid, sid, start, count


def _zero_spmem_range(zbuf, acc, sid):
  for t in range(4):
    pltpu.sync_copy(zbuf, acc.at[pl.ds(sid * RSUB + t * CH, CH)])
  pltpu.sync_copy(zbuf.at[pl.ds(0, RSUB - 4 * CH)],
                  acc.at[pl.ds(sid * RSUB + 4 * CH, RSUB - 4 * CH)])

  @pl.when(sid == 0)
  def _():
    pltpu.sync_copy(zbuf.at[pl.ds(0, TAIL)], acc.at[pl.ds(NS * RSUB, TAIL)])


def _export_spmem_range(acc, out_hbm, cid, sid):
  pltpu.sync_copy(acc.at[pl.ds(sid * RSUB, RSUB)],
                  out_hbm.at[cid, pl.ds(sid * RSUB, RSUB)])

  @pl.when(sid == 0)
  def _():
    pltpu.sync_copy(acc.at[pl.ds(NS * RSUB, TAIL)],
                    out_hbm.at[cid, pl.ds(NS * RSUB, TAIL)])


def _deg_body(dst_hbm, degp_hbm, ones_b, zero_b, idx_r, acc, sem_i, sem_s):
  cid, sid, start, count = _worker_ids()

  one_row = jnp.where(lax.iota(jnp.int32, 16) == 0, 1.0, 0.0)
  zro_row = jnp.zeros((16,), jnp.float32)

  def fill(i, _):
    ones_b[i] = one_row
    zero_b[i] = zro_row
    return _
  lax.fori_loop(0, CH, fill, None)

  _zero_spmem_range(zero_b, acc, sid)
  plsc.subcore_barrier()

  def _idx(c, slot):
    return (dst_hbm.at[pl.ds((start + c) * CH, CH)], idx_r.at[slot],
            sem_i.at[slot])

  def _sct(slot):
    return (ones_b, acc.at[idx_r.at[slot]], sem_s.at[slot])

  for c in range(4):
    pltpu.async_copy(*_idx(c, c))

  def step(t, _):
    for u in range(8):
      i = 8 * t + u

      @pl.when(i < count)
      def _():
        pltpu.make_async_copy(*_idx(i, u)).wait()
        s, d, m = _sct(u)
        pltpu.async_copy(s, d, m, add=True)

        @pl.when(i >= 4)
        def _():
          s2, d2, m2 = _sct((u + 4) % 8)
          pltpu.make_async_copy(s2, d2, m2).wait()

        @pl.when(i + 4 < count)
        def _():
          pltpu.async_copy(*_idx(i + 4, (u + 4) % 8))
    return _
  lax.fori_loop(0, (MAXC + 7) // 8, step, None)

  for cnt in (BASE_CNT, MAXC):
    @pl.when(count == cnt)
    def _():
      for k in range(1, 5):
        s, d, m = _sct((cnt - k) % 8)
        pltpu.make_async_copy(s, d, m).wait()

  plsc.subcore_barrier()
  _export_spmem_range(acc, degp_hbm, cid, sid)


_deg_kernel = pl.kernel(
    _deg_body,
    out_type=jax.ShapeDtypeStruct((NC, N_NODES, 16), jnp.float32),
    mesh=_mesh,
    compiler_params=_sc_params,
    scratch_types=[
        pltpu.VMEM((CH, 16), jnp.float32),
        pltpu.VMEM((CH, 16), jnp.float32),
        pltpu.VMEM((8, CH), jnp.int32),
        pltpu.VMEM_SHARED((N_NODES, 16), jnp.float32),
        pltpu.SemaphoreType.DMA((8,)),
        pltpu.SemaphoreType.DMA((8,)),
    ],
)


def _k2_body(x_ref, w1_ref, degp_ref, y_ref):
  deg = 1.0 + jnp.sum(degp_ref[...], axis=(0, 2))
  dinv = lax.rsqrt(deg)
  xw = jnp.dot(x_ref[...], w1_ref[...], preferred_element_type=jnp.float32)
  y_ref[...] = xw * dinv[:, None]


def _scatter_body(y_hbm, src_hbm, dst_hbm, aggp_hbm, rows, idx_r, acc,
                  sem_i, sem_g, sem_s):
  cid, sid, start, count = _worker_ids()

  zro = jnp.zeros((16,), jnp.float32)

  def zfill(i, _):
    for k in range(8):
      rows[0, i, pl.ds(k * 16, 16)] = zro
    return _
  lax.fori_loop(0, CH, zfill, None)

  _zero_spmem_range(rows.at[0], acc, sid)
  plsc.subcore_barrier()

  def _idx(c, slot):
    g = (start + c) * CH
    return ((src_hbm.at[pl.ds(g, CH)], idx_r.at[slot, 0], sem_i.at[slot]),
            (dst_hbm.at[pl.ds(g, CH)], idx_r.at[slot, 1], sem_i.at[slot]))

  def _gat(islot, rslot):
    return (y_hbm.at[idx_r.at[islot, 0]], rows.at[rslot], sem_g.at[rslot])

  def _sct(islot, rslot):
    return (rows.at[rslot], acc.at[idx_r.at[islot, 1]], sem_s.at[rslot])

  def fire_idx(c, slot):
    a, b = _idx(c, slot)
    pltpu.async_copy(*a)
    pltpu.async_copy(*b)

  def wait_idx(c, slot):
    a, b = _idx(c, slot)
    pltpu.make_async_copy(*a).wait()
    pltpu.make_async_copy(*b).wait()

  for c in range(4):
    fire_idx(c, c)
  for c in range(2):
    wait_idx(c, c)
    pltpu.async_copy(*_gat(c, c))

  def step(t, _):
    for u in range(6):
      i = 6 * t + u
      ru = u % 3

      @pl.when(i < count)
      def _():
        g = _gat(u, ru)
        pltpu.make_async_copy(*g).wait()
        s, d, m = _sct(u, ru)
        pltpu.async_copy(s, d, m, add=True)

        @pl.when(i >= 1)
        def _():
          s2, d2, m2 = _sct((u + 5) % 6, (ru + 2) % 3)
          pltpu.make_async_copy(s2, d2, m2).wait()

        @pl.when(i + 2 < count)
        def _():
          wait_idx(i + 2, (u + 2) % 6)
          pltpu.async_copy(*_gat((u + 2) % 6, (ru + 2) % 3))

        @pl.when(i + 4 < count)
        def _():
          fire_idx(i + 4, (u + 4) % 6)
    return _
  lax.fori_loop(0, (MAXC + 5) // 6, step, None)

  for cnt in (BASE_CNT, MAXC):
    @pl.when(count == cnt)
    def _():
      s, d, m = _sct((cnt - 1) % 6, (cnt - 1) % 3)
      pltpu.make_async_copy(s, d, m).wait()

  plsc.subcore_barrier()
  _export_spmem_range(acc, aggp_hbm, cid, sid)


_scatter_kernel = pl.kernel(
    _scatter_body,
    out_type=jax.ShapeDtypeStruct((NC, N_NODES, D), jnp.float32),
    mesh=_mesh,
    compiler_params=_sc_params,
    scratch_types=[
        pltpu.VMEM((3, CH, D), jnp.float32),
        pltpu.VMEM((6, 2, CH), jnp.int32),
        pltpu.VMEM_SHARED((N_NODES, D), jnp.float32),
        pltpu.SemaphoreType.DMA((6,)),
        pltpu.SemaphoreType.DMA((3,)),
        pltpu.SemaphoreType.DMA((3,)),
    ],
)


def _k4a_body(aggp_ref, y_ref, degp_ref, b1_ref, wa_ref, a_ref, b_ref):
  deg = 1.0 + jnp.sum(degp_ref[...], axis=(0, 2))
  dinv = lax.rsqrt(deg)
  pre = dinv[:, None] * (aggp_ref[0] + aggp_ref[1] + y_ref[...]) + b1_ref[...]
  h = jnp.maximum(pre, 0.0)
  a_ref[...] = jnp.dot(h, wa_ref[0:D, :], preferred_element_type=jnp.float32)
  b_ref[...] = jnp.dot(h, wa_ref[D:2 * D, :],
                       preferred_element_type=jnp.float32)


def _k4b_body(ea2_ref, we2_ref, wo2_ref, bae2_ref, bao2_ref, c_ref):
  ce = jnp.dot(ea2_ref[...], we2_ref[...],
               preferred_element_type=jnp.float32) + bae2_ref[...]
  co = jnp.dot(ea2_ref[...], wo2_ref[...],
               preferred_element_type=jnp.float32) + bao2_ref[...]
  ue = jax.lax.bitcast_convert_type(ce, jnp.uint32)
  uo = jax.lax.bitcast_convert_type(co, jnp.uint32)
  lo = (ue + jnp.uint32(0x7FFF) + ((ue >> 16) & jnp.uint32(1))) >> 16
  hi = ((uo + jnp.uint32(0x7FFF) + ((uo >> 16) & jnp.uint32(1)))
        & jnp.uint32(0xFFFF0000))
  c_ref[...] = lo | hi


def _edge_body(a_hbm, b_hbm, c_hbm, src_hbm, dst_hbm, wb_hbm, bb_hbm,
               out_hbm, a2, b2, c2, idx_r, wb_v, bb_v, out_b,
               sem_a, sem_b, sem_c, sem_i):
  cid, sid, start, count = _worker_ids()

  pltpu.sync_copy(wb_hbm, wb_v)
  pltpu.sync_copy(bb_hbm, bb_v)
  wbk = [wb_v[pl.ds(k * 16, 16)] for k in range(8)]
  bbv = bb_v[...]
  lane = lax.iota(jnp.int32, 16)

  def _idx(c, slot):
    g = (start + c) * CH
    return ((src_hbm.at[pl.ds(g, CH)], idx_r.at[slot, 0], sem_i.at[slot]),
            (dst_hbm.at[pl.ds(g, CH)], idx_r.at[slot, 1], sem_i.at[slot]))

  def _abc(c, islot, p):
    return ((a_hbm.at[idx_r.at[islot, 0]], a2.at[p], sem_a.at[p]),
            (b_hbm.at[idx_r.at[islot, 1]], b2.at[p], sem_b.at[p]),
            (c_hbm.at[pl.ds((start + c) * (CH // 2), CH // 2)], c2.at[p],
             sem_c.at[p]))

  def fire_idx(c, slot):
    a, b = _idx(c, slot)
    pltpu.async_copy(*a)
    pltpu.async_copy(*b)

  def wait_idx(c, slot):
    a, b = _idx(c, slot)
    pltpu.make_async_copy(*a).wait()
    pltpu.make_async_copy(*b).wait()

  def fire_abc(c, islot, p):
    for triple in _abc(c, islot, p):
      pltpu.async_copy(*triple)

  def wait_abc(c, islot, p):
    for triple in _abc(c, islot, p):
      pltpu.make_async_copy(*triple).wait()

  fire_idx(0, 0)
  fire_idx(1, 1)
  wait_idx(0, 0)
  fire_abc(0, 0, 0)
  wait_idx(1, 1)
  fire_abc(1, 1, 1)
  fire_idx(2, 2)
  fire_idx(3, 3)

  def step(t, _):
    for u in range(4):
      i = 4 * t + u
      p = u % 2

      @pl.when(i < count)
      def _():
        wait_abc(i, u, p)

        def group(g, _):
          out_vec = jnp.zeros((16,), jnp.float32)
          for l in range(16):
            j = g * 16 + l
            acc = jnp.zeros((16,), jnp.float32)
            for kp in range(4):
              cp = c2[p, g * 8 + l // 2, pl.ds((l % 2) * 64 + kp * 16, 16)]
              clo = plsc.bitcast(cp << 16, jnp.float32)
              chi = plsc.bitcast(cp & jnp.uint32(0xFFFF0000), jnp.float32)
              for k, cv in ((2 * kp, clo), (2 * kp + 1, chi)):
                sl = pl.ds(k * 16, 16)
                v = a2[p, j, sl] + b2[p, j, sl] + cv
                acc = acc + jnp.maximum(v, 0.0) * wbk[k]
            out_vec = jnp.where(lane == l, jnp.sum(acc), out_vec)
          sv = out_vec + bbv
          out_b[pl.ds(i * CH + g * 16, 16)] = 1.0 / (1.0 + jnp.exp(-sv))
          return _
        lax.fori_loop(0, CH // 16, group, None)

        @pl.when(i + 2 < count)
        def _():
          wait_idx(i + 2, (u + 2) % 4)
          fire_abc(i + 2, (u + 2) % 4, p)

        @pl.when(i + 4 < count)
        def _():
          fire_idx(i + 4, u)
    return _
  lax.fori_loop(0, (MAXC + 3) // 4, step, None)

  pltpu.sync_copy(out_b.at[pl.ds(0, BASE_CNT * CH)],
                  out_hbm.at[pl.ds(start * CH, BASE_CNT * CH)])

  @pl.when(count == MAXC)
  def _():
    pltpu.sync_copy(out_b.at[pl.ds(BASE_CNT * CH, CH)],
                    out_hbm.at[pl.ds((start + BASE_CNT) * CH, CH)])


_edge_kernel = pl.kernel(
    _edge_body,
    out_type=jax.ShapeDtypeStruct((N_EDGES,), jnp.float32),
    mesh=_mesh,
    compiler_params=_sc_params,
    scratch_types=[
        pltpu.VMEM((2, CH, D), jnp.float32),
        pltpu.VMEM((2, CH, D), jnp.float32),
        pltpu.VMEM((2, CH // 2, D), jnp.uint32),
        pltpu.VMEM((4, 2, CH), jnp.int32),
        pltpu.VMEM((D,), jnp.float32),
        pltpu.VMEM((16,), jnp.float32),
        pltpu.VMEM((MAXC * CH,), jnp.float32),
        pltpu.SemaphoreType.DMA((2,)),
        pltpu.SemaphoreType.DMA((2,)),
        pltpu.SemaphoreType.DMA((2,)),
        pltpu.SemaphoreType.DMA((4,)),
    ],
)


def kernel(x, edge_index, edge_attr, W1, b1, Wa, ba, Wb, bb):
  src = edge_index[0].astype(jnp.int32)
  dst = edge_index[1].astype(jnp.int32)

  degp = _deg_kernel(dst)

  y = pl.pallas_call(
      _k2_body,
      out_shape=jax.ShapeDtypeStruct((N_NODES, D), jnp.float32),
  )(x, W1, degp)

  aggp = _scatter_kernel(y, src, dst)

  a_tab, b_tab = pl.pallas_call(
      _k4a_body,
      out_shape=(jax.ShapeDtypeStruct((N_NODES, D), jnp.float32),
                 jax.ShapeDtypeStruct((N_NODES, D), jnp.float32)),
  )(aggp, y, degp, b1.reshape(1, D), Wa)

  qe = np.concatenate([np.arange(32 * kp, 32 * kp + 16) for kp in range(4)])
  qo = qe + 16
  wae = Wa[2 * D:2 * D + EDGE_DIM, :]
  wae_e, wae_o = wae[:, qe], wae[:, qo]
  zer = jnp.zeros((EDGE_DIM, D // 2), jnp.float32)
  we2 = jnp.concatenate(
      [jnp.concatenate([wae_e, zer], 1), jnp.concatenate([zer, wae_e], 1)], 0)
  wo2 = jnp.concatenate(
      [jnp.concatenate([wae_o, zer], 1), jnp.concatenate([zer, wae_o], 1)], 0)
  bae2 = jnp.concatenate([ba[qe], ba[qe]]).reshape(1, D)
  bao2 = jnp.concatenate([ba[qo], ba[qo]]).reshape(1, D)
  ea2 = edge_attr.reshape(N_EDGES // 2, 2 * EDGE_DIM)

  blk = 20000
  c_tab = pl.pallas_call(
      _k4b_body,
      grid=(N_EDGES // blk,),
      in_specs=[
          pl.BlockSpec((blk // 2, 2 * EDGE_DIM), lambda i: (i, 0)),
          pl.BlockSpec((2 * EDGE_DIM, D), lambda i: (0, 0)),
          pl.BlockSpec((2 * EDGE_DIM, D), lambda i: (0, 0)),
          pl.BlockSpec((1, D), lambda i: (0, 0)),
          pl.BlockSpec((1, D), lambda i: (0, 0)),
      ],
      out_specs=pl.BlockSpec((blk // 2, D), lambda i: (i, 0)),
      out_shape=jax.ShapeDtypeStruct((N_EDGES // 2, D), jnp.uint32),
  )(ea2, we2, wo2, bae2, bao2)

  scores = _edge_kernel(a_tab, b_tab, c_tab, src, dst, Wb[:, 0],
                        jnp.broadcast_to(bb, (16,)))
  return scores
